# baseline (speedup 1.0000x reference)
import functools

import jax
import jax.numpy as jnp
from jax import lax
from jax.experimental import pallas as pl
from jax.experimental.pallas import tpu as pltpu
from jax.experimental.pallas import tpu_sc as plsc

B, H, S, E = 8, 16, 2048, 128


_HB = 8


def _tc_copy_body(n_ref, t_ref, cache_ref, out_ref):
    out_ref[...] = cache_ref[...]
    slot = n_ref[0] + 1
    out_ref[0, :, pl.ds(slot, 1), :] = t_ref[0, :, :, :]


def _tc_copy(n_arr, t, cache):
    cache_spec = pl.BlockSpec((1, _HB, S, E), lambda b, h: (b, h, 0, 0))
    t_spec = pl.BlockSpec((1, _HB, 1, E), lambda b, h: (b, h, 0, 0))
    return pl.pallas_call(
        _tc_copy_body,
        grid=(B, H // _HB),
        out_shape=jax.ShapeDtypeStruct(cache.shape, cache.dtype),
        in_specs=[pl.BlockSpec(memory_space=pltpu.MemorySpace.SMEM),
                  t_spec, cache_spec],
        out_specs=cache_spec,
        compiler_params=pltpu.CompilerParams(
            dimension_semantics=("parallel", "parallel")),
    )(n_arr, t, cache)


_NC, _NS = 2, 16
_NW = _NC * _NS
_HPW = (B * H) // _NW
_CHUNK = 256
_SCHUNKS = S // _CHUNK


def _sc_copy_body(slot_hbm, vt_hbm, vc_hbm, out_hbm,
                  buf, rowbuf, slot_idx, insems, outsems, rowsem):
    cc = lax.axis_index("c")
    ss = lax.axis_index("s")
    wid = ss * _NC + cc
    pltpu.sync_copy(slot_hbm, slot_idx)

    def refs(t):
        j, ci = divmod(t, _SCHUNKS)
        head = wid * _HPW + j
        b = head // H
        h = head % H
        src = vc_hbm.at[b, h, pl.ds(ci * _CHUNK, _CHUNK), :]
        dst = out_hbm.at[b, h, pl.ds(ci * _CHUNK, _CHUNK), :]
        return src, dst

    total = _HPW * _SCHUNKS
    in_copies = [None] * total
    out_copies = [None] * total
    src0, _ = refs(0)
    in_copies[0] = pltpu.make_async_copy(src0, buf.at[0], insems.at[0])
    in_copies[0].start()
    for t in range(total):
        bi = t % 2
        if t + 1 < total:
            if t >= 1:
                out_copies[t - 1].wait()
            srcn, _ = refs(t + 1)
            nbi = (t + 1) % 2
            in_copies[t + 1] = pltpu.make_async_copy(
                srcn, buf.at[nbi], insems.at[nbi])
            in_copies[t + 1].start()
        in_copies[t].wait()
        _, dst = refs(t)
        out_copies[t] = pltpu.make_async_copy(buf.at[bi], dst, outsems.at[bi])
        out_copies[t].start()
    out_copies[total - 2].wait()
    out_copies[total - 1].wait()

    for j in range(_HPW):
        head = wid * _HPW + j
        b = head // H
        h = head % H
        cin = pltpu.make_async_copy(vt_hbm.at[b, h], rowbuf, rowsem)
        cin.start()
        cin.wait()
        cout = pltpu.make_async_copy(
            rowbuf, out_hbm.at[b, h].at[slot_idx], rowsem)
        cout.start()
        cout.wait()


def _sc_copy(n_arr, t, cache):
    mesh = plsc.VectorSubcoreMesh(core_axis_name="c", subcore_axis_name="s")
    fn = functools.partial(
        pl.kernel,
        mesh=mesh,
        out_type=jax.ShapeDtypeStruct(cache.shape, cache.dtype),
        scratch_types=[
            pltpu.VMEM((2, _CHUNK, E), jnp.float32),
            pltpu.VMEM((1, E), jnp.float32),
            pltpu.VMEM((1,), jnp.int32),
            pltpu.SemaphoreType.DMA((2,)),
            pltpu.SemaphoreType.DMA((2,)),
            pltpu.SemaphoreType.DMA,
        ],
    )(_sc_copy_body)
    return fn(n_arr, t, cache)


def kernel(k_t, v_t, k_cache, v_cache, n_cached):
    n_arr = jnp.asarray(n_cached, jnp.int32).reshape(1)
    slot_arr = n_arr + 1
    new_k = _tc_copy(n_arr, k_t, k_cache)
    new_v = _sc_copy(slot_arr, v_t, v_cache)
    return (new_k, new_v)


# speedup vs baseline: 1.0125x; 1.0125x over previous
import functools

import jax
import jax.numpy as jnp
from jax import lax
from jax.experimental import pallas as pl
from jax.experimental.pallas import tpu as pltpu
from jax.experimental.pallas import tpu_sc as plsc

B, H, S, E = 8, 16, 2048, 128
_SPLIT = 768


_HB = 8


def _tc_copy_body(n_ref, t_ref, cache_ref, out_ref):
    out_ref[...] = cache_ref[...]
    slot = n_ref[0] + 1
    out_ref[0, :, pl.ds(slot, 1), :] = t_ref[0, :, :, :]


def _tc_copy(n_arr, t, cache):
    cache_spec = pl.BlockSpec((1, _HB, S, E), lambda b, h: (b, h, 0, 0))
    t_spec = pl.BlockSpec((1, _HB, 1, E), lambda b, h: (b, h, 0, 0))
    return pl.pallas_call(
        _tc_copy_body,
        grid=(B, H // _HB),
        out_shape=jax.ShapeDtypeStruct(cache.shape, cache.dtype),
        in_specs=[pl.BlockSpec(memory_space=pltpu.MemorySpace.SMEM),
                  t_spec, cache_spec],
        out_specs=cache_spec,
        compiler_params=pltpu.CompilerParams(
            dimension_semantics=("parallel", "parallel")),
    )(n_arr, t, cache)


_SB = 256
_S_TC = S - _SPLIT


def _tc_finish_body(n_ref, t_ref, cache_ref, part_ref, out_ref):
    del part_ref
    out_ref[...] = cache_ref[...]
    slot = n_ref[0] + 1
    sk = pl.program_id(1)
    base = _SPLIT + sk * _SB

    @pl.when(jnp.logical_and(slot >= base, slot < base + _SB))
    def _():
        out_ref[0, :, pl.ds(slot - base, 1), :] = t_ref[0, :, :, :]


def _tc_finish(n_arr, t, part, cache):
    cache_spec = pl.BlockSpec(
        (1, H, _SB, E), lambda b, k: (b, 0, k + _SPLIT // _SB, 0))
    t_spec = pl.BlockSpec((1, H, 1, E), lambda b, k: (b, 0, 0, 0))
    return pl.pallas_call(
        _tc_finish_body,
        grid=(B, _S_TC // _SB),
        out_shape=jax.ShapeDtypeStruct(cache.shape, cache.dtype),
        in_specs=[pl.BlockSpec(memory_space=pltpu.MemorySpace.SMEM),
                  t_spec,
                  cache_spec,
                  pl.BlockSpec(memory_space=pltpu.MemorySpace.HBM)],
        out_specs=cache_spec,
        input_output_aliases={2: 0},
        compiler_params=pltpu.CompilerParams(
            dimension_semantics=("parallel", "parallel")),
    )(n_arr, t, part, cache)


_NC, _NS = 2, 16
_NW = _NC * _NS
_HPW = (B * H) // _NW
_CHUNK = 256
_SCHUNKS = _SPLIT // _CHUNK


def _sc_copy_body(slot_hbm, vt_hbm, vc_hbm, out_hbm,
                  buf, rowbuf, slot_idx, insems, outsems, rowsem):
    cc = lax.axis_index("c")
    ss = lax.axis_index("s")
    wid = ss * _NC + cc
    pltpu.sync_copy(slot_hbm, slot_idx)

    def refs(t):
        j, ci = divmod(t, _SCHUNKS)
        head = wid * _HPW + j
        b = head // H
        h = head % H
        src = vc_hbm.at[b, h, pl.ds(ci * _CHUNK, _CHUNK), :]
        dst = out_hbm.at[b, h, pl.ds(ci * _CHUNK, _CHUNK), :]
        return src, dst

    total = _HPW * _SCHUNKS
    in_copies = [None] * total
    out_copies = [None] * total
    src0, _ = refs(0)
    in_copies[0] = pltpu.make_async_copy(src0, buf.at[0], insems.at[0])
    in_copies[0].start()
    for t in range(total):
        bi = t % 2
        if t + 1 < total:
            if t >= 1:
                out_copies[t - 1].wait()
            srcn, _ = refs(t + 1)
            nbi = (t + 1) % 2
            in_copies[t + 1] = pltpu.make_async_copy(
                srcn, buf.at[nbi], insems.at[nbi])
            in_copies[t + 1].start()
        in_copies[t].wait()
        _, dst = refs(t)
        out_copies[t] = pltpu.make_async_copy(buf.at[bi], dst, outsems.at[bi])
        out_copies[t].start()
    out_copies[total - 2].wait()
    out_copies[total - 1].wait()

    for j in range(_HPW):
        head = wid * _HPW + j
        b = head // H
        h = head % H
        cin = pltpu.make_async_copy(vt_hbm.at[b, h], rowbuf, rowsem)
        cin.start()
        cin.wait()
        cout = pltpu.make_async_copy(
            rowbuf, out_hbm.at[b, h].at[slot_idx], rowsem)
        cout.start()
        cout.wait()


def _sc_copy(slot_arr, t, cache):
    mesh = plsc.VectorSubcoreMesh(core_axis_name="c", subcore_axis_name="s")
    fn = functools.partial(
        pl.kernel,
        mesh=mesh,
        out_type=jax.ShapeDtypeStruct(cache.shape, cache.dtype),
        scratch_types=[
            pltpu.VMEM((2, _CHUNK, E), jnp.float32),
            pltpu.VMEM((1, E), jnp.float32),
            pltpu.VMEM((1,), jnp.int32),
            pltpu.SemaphoreType.DMA((2,)),
            pltpu.SemaphoreType.DMA((2,)),
            pltpu.SemaphoreType.DMA,
        ],
    )(_sc_copy_body)
    return fn(slot_arr, t, cache)


def kernel(k_t, v_t, k_cache, v_cache, n_cached):
    n_arr = jnp.asarray(n_cached, jnp.int32).reshape(1)
    slot_arr = n_arr + 1
    v_part = _sc_copy(slot_arr, v_t, v_cache)
    new_k = _tc_copy(n_arr, k_t, k_cache)
    new_v = _tc_finish(n_arr, v_t, v_part, v_cache)
    return (new_k, new_v)


# speedup vs baseline: 1.1650x; 1.1506x over previous
import jax
import jax.numpy as jnp
from jax.experimental import pallas as pl
from jax.experimental.pallas import tpu as pltpu

B, H, S, E = 8, 16, 2048, 128
_HB = 4


def _kvcache_kernel(n_ref, k_t, v_t, k_cache, v_cache, out_k, out_v):
    out_k[...] = k_cache[...]
    out_v[...] = v_cache[...]
    slot = n_ref[0] + 1
    out_k[0, :, pl.ds(slot, 1), :] = k_t[0, :, :, :]
    out_v[0, :, pl.ds(slot, 1), :] = v_t[0, :, :, :]


def kernel(k_t, v_t, k_cache, v_cache, n_cached):
    n_arr = jnp.asarray(n_cached, jnp.int32).reshape(1)
    cache_spec = pl.BlockSpec((1, _HB, S, E), lambda b, h: (b, h, 0, 0))
    t_spec = pl.BlockSpec((1, _HB, 1, E), lambda b, h: (b, h, 0, 0))
    return pl.pallas_call(
        _kvcache_kernel,
        grid=(B, H // _HB),
        out_shape=(jax.ShapeDtypeStruct(k_cache.shape, k_cache.dtype),
                   jax.ShapeDtypeStruct(v_cache.shape, v_cache.dtype)),
        in_specs=[pl.BlockSpec(memory_space=pltpu.MemorySpace.SMEM),
                  t_spec, t_spec, cache_spec, cache_spec],
        out_specs=(cache_spec, cache_spec),
        compiler_params=pltpu.CompilerParams(
            dimension_semantics=("parallel", "parallel")),
    )(n_arr, k_t, v_t, k_cache, v_cache)
